# baseline (speedup 1.0000x reference)
import functools

import jax
import jax.numpy as jnp
from jax import lax
from jax.experimental import pallas as pl
from jax.experimental.pallas import tpu as pltpu
from jax.experimental.pallas import tpu_sc as plsc

VOCAB = 30522
HIDDEN = 768
LN_EPS = 1e-12
B, S = 1024, 512
N = B * S

NC, NS, L = 2, 16, 16
NW = NC * NS
TPW = N // NW
C = 16
NCHUNK = TPW // C
UNROLL = 4

_mesh = plsc.VectorSubcoreMesh(core_axis_name="c", subcore_axis_name="s")


@functools.partial(
    pl.kernel,
    mesh=_mesh,
    out_type=jax.ShapeDtypeStruct((N, HIDDEN), jnp.float32),
    compiler_params=pltpu.CompilerParams(
        use_tc_tiling_on_sc=False, needs_layout_passes=False
    ),
    scratch_types=[
        pltpu.VMEM((TPW,), jnp.int32),
        pltpu.VMEM((TPW,), jnp.int32),
        pltpu.VMEM((TPW,), jnp.int32),
        pltpu.VMEM((2, HIDDEN), jnp.float32),
        pltpu.VMEM((HIDDEN,), jnp.float32),
        pltpu.VMEM((HIDDEN,), jnp.float32),
        pltpu.VMEM((C, HIDDEN), jnp.float32),
        pltpu.VMEM((C, HIDDEN), jnp.float32),
        pltpu.VMEM((C, HIDDEN), jnp.float32),
        pltpu.SemaphoreType.DMA,
        pltpu.SemaphoreType.DMA,
        pltpu.SemaphoreType.DMA,
    ],
)
def _sc_embed(ids_hbm, pids_hbm, tids_hbm, wtab, ptab, ttab, gam, bet,
              out_hbm,
              widx, pidx, tidx, ttb, gb, bb, wbuf, pbuf, obuf,
              sem_w, sem_p, sem_o):
    wid = lax.axis_index("s") * NC + lax.axis_index("c")
    base = wid * TPW

    pltpu.sync_copy(ids_hbm.at[pl.ds(base, TPW)], widx)
    pltpu.sync_copy(pids_hbm.at[pl.ds(base, TPW)], pidx)
    pltpu.sync_copy(tids_hbm.at[pl.ds(base, TPW)], tidx)
    pltpu.sync_copy(ttab, ttb)
    pltpu.sync_copy(gam, gb)
    pltpu.sync_copy(bet, bb)

    lanes = lax.iota(jnp.int32, L)

    def chunk(g, carry):
        c0 = g * C
        wvec = widx[pl.ds(c0, C)]
        pvec = pidx[pl.ds(c0, C)]
        tvec = tidx[pl.ds(c0, C)]
        cw = pltpu.async_copy(wtab.at[wvec], wbuf, sem_w)
        cp = pltpu.async_copy(ptab.at[pvec], pbuf, sem_p)
        cw.wait()
        cp.wait()

        def pass1(jj, sq):
            s, q = sq
            for dj in range(UNROLL):
                j = jj * UNROLL + dj
                jsp = jnp.full((L,), j, jnp.int32)
                wv = plsc.load_gather(wbuf, [lanes, jsp])
                pv = plsc.load_gather(pbuf, [lanes, jsp])
                tv = plsc.load_gather(ttb, [tvec, jsp])
                x = wv + pv + tv
                plsc.store_scatter(wbuf, [lanes, jsp], x)
                s = s + x
                q = q + x * x
            return (s, q)

        zero = jnp.zeros((L,), jnp.float32)
        s, q = lax.fori_loop(0, HIDDEN // UNROLL, pass1, (zero, zero))
        mean = s * (1.0 / HIDDEN)
        var = q * (1.0 / HIDDEN) - mean * mean
        y = var + LN_EPS
        ib = plsc.bitcast(y, jnp.int32)
        ib = 0x5F3759DF - lax.shift_right_logical(ib, 1)
        r = plsc.bitcast(ib, jnp.float32)
        for _ in range(4):
            r = r * (1.5 - 0.5 * y * r * r)

        def pass2(jj, c):
            for dj in range(UNROLL):
                j = jj * UNROLL + dj
                jsp = jnp.full((L,), j, jnp.int32)
                xv = plsc.load_gather(wbuf, [lanes, jsp])
                gv = plsc.load_gather(gb, [jsp])
                bv = plsc.load_gather(bb, [jsp])
                yv = (xv - mean) * r * gv + bv
                plsc.store_scatter(obuf, [lanes, jsp], yv)
            return c

        lax.fori_loop(0, HIDDEN // UNROLL, pass2, 0)
        co = pltpu.async_copy(obuf, out_hbm.at[pl.ds(base + c0, C)], sem_o)
        co.wait()
        return carry

    lax.fori_loop(0, NCHUNK, chunk, 0)


def kernel(input_ids, token_type_ids, position_ids, word_embeddings,
           position_embeddings, token_type_embeddings, ln_gamma, ln_beta):
    ids = input_ids.reshape(-1).astype(jnp.int32)
    pids = position_ids.reshape(-1).astype(jnp.int32)
    tids = token_type_ids.reshape(-1).astype(jnp.int32)
    out = _sc_embed(ids, pids, tids, word_embeddings, position_embeddings,
                    token_type_embeddings, ln_gamma, ln_beta)
    return out.reshape(B, S, HIDDEN)


# speedup vs baseline: 1.2116x; 1.2116x over previous
import functools

import jax
import jax.numpy as jnp
from jax import lax
from jax.experimental import pallas as pl
from jax.experimental.pallas import tpu as pltpu
from jax.experimental.pallas import tpu_sc as plsc

VOCAB = 30522
HIDDEN = 768
MAX_POS = 512
TYPE_VOCAB = 2
LN_EPS = 1e-12
B, S = 1024, 512
N = B * S

NC, NS, L = 2, 16, 16
NW = NC * NS
TPW = N // NW
C = 16
NCHUNK = TPW // C
UNROLL = 8

_mesh = plsc.VectorSubcoreMesh(core_axis_name="c", subcore_axis_name="s")


@functools.partial(
    pl.kernel,
    mesh=_mesh,
    out_type=jax.ShapeDtypeStruct((N, HIDDEN), jnp.float32),
    compiler_params=pltpu.CompilerParams(
        use_tc_tiling_on_sc=False, needs_layout_passes=False
    ),
    scratch_types=[
        pltpu.VMEM((TPW,), jnp.int32),
        pltpu.VMEM((TPW,), jnp.int32),
        pltpu.VMEM((HIDDEN,), jnp.float32),
        pltpu.VMEM((HIDDEN,), jnp.float32),
        pltpu.VMEM((C, HIDDEN), jnp.float32),
        pltpu.VMEM((C, HIDDEN), jnp.float32),
        pltpu.VMEM((C, HIDDEN), jnp.float32),
        pltpu.VMEM((C, HIDDEN), jnp.float32),
        pltpu.VMEM((C, HIDDEN), jnp.float32),
        pltpu.VMEM((C, HIDDEN), jnp.float32),
        pltpu.SemaphoreType.DMA,
        pltpu.SemaphoreType.DMA,
        pltpu.SemaphoreType.DMA,
        pltpu.SemaphoreType.DMA,
        pltpu.SemaphoreType.DMA,
        pltpu.SemaphoreType.DMA,
    ],
)
def _sc_embed(ids_hbm, cids_hbm, wtab, ctab, gam, bet,
              out_hbm,
              widx, cidx, gvm, bvm,
              wb0, wb1, cb0, cb1, ob0, ob1,
              sw0, sw1, sc0, sc1, so0, so1):
    wid = lax.axis_index("s") * NC + lax.axis_index("c")
    base = wid * TPW

    pltpu.sync_copy(ids_hbm.at[pl.ds(base, TPW)], widx)
    pltpu.sync_copy(cids_hbm.at[pl.ds(base, TPW)], cidx)
    pltpu.sync_copy(gam, gvm)
    pltpu.sync_copy(bet, bvm)

    lanes = lax.iota(jnp.int32, L)
    slots = ((wb0, cb0, ob0, sw0, sc0, so0),
             (wb1, cb1, ob1, sw1, sc1, so1))

    def issue(g, slot):
        wb, cb, _, sw, sc_, _ = slots[slot]
        c0 = g * C
        wvec = widx[pl.ds(c0, C)]
        cvec = cidx[pl.ds(c0, C)]
        pltpu.async_copy(wtab.at[wvec], wb, sw)
        pltpu.async_copy(ctab.at[cvec], cb, sc_)

    issue(0, 0)

    def outer(g2, carry):
        for sl in range(2):
            wb, cb, ob, sw, sc_, so = slots[sl]
            g = g2 * 2 + sl

            @pl.when(g + 1 < NCHUNK)
            def _():
                issue(g + 1, 1 - sl)

            dummy = widx[pl.ds(0, C)]
            pltpu.make_async_copy(wtab.at[dummy], wb, sw).wait()
            pltpu.make_async_copy(ctab.at[dummy], cb, sc_).wait()

            def pass1(jj, carry1):
                s0, q0, s1, q1, jsp = carry1
                for dj in range(UNROLL):
                    wv = plsc.load_gather(wb, [lanes, jsp])
                    cv = plsc.load_gather(cb, [lanes, jsp])
                    x = wv + cv
                    plsc.store_scatter(wb, [lanes, jsp], x)
                    if dj % 2 == 0:
                        s0 = s0 + x
                        q0 = q0 + x * x
                    else:
                        s1 = s1 + x
                        q1 = q1 + x * x
                    jsp = jsp + 1
                return (s0, q0, s1, q1, jsp)

            zero = jnp.zeros((L,), jnp.float32)
            jsp0 = jnp.zeros((L,), jnp.int32)
            s0, q0, s1, q1, _ = lax.fori_loop(
                0, HIDDEN // UNROLL, pass1, (zero, zero, zero, zero, jsp0))
            s = s0 + s1
            q = q0 + q1
            mean = s * (1.0 / HIDDEN)
            var = q * (1.0 / HIDDEN) - mean * mean
            y = var + LN_EPS
            ib = plsc.bitcast(y, jnp.int32)
            ib = 0x5F3759DF - lax.shift_right_logical(ib, 1)
            r = plsc.bitcast(ib, jnp.float32)
            for _ in range(4):
                r = r * (1.5 - 0.5 * y * r * r)
            mr = mean * r

            @pl.when(g >= 2)
            def _():
                pltpu.make_async_copy(
                    ob, out_hbm.at[pl.ds(base, C)], so).wait()

            def pass2(jj, carry2):
                jsp = carry2
                for dj in range(UNROLL):
                    xv = plsc.load_gather(wb, [lanes, jsp])
                    gv = plsc.load_gather(gvm, [jsp])
                    bv = plsc.load_gather(bvm, [jsp])
                    yv = (xv * r - mr) * gv + bv
                    plsc.store_scatter(ob, [lanes, jsp], yv)
                    jsp = jsp + 1
                return jsp

            lax.fori_loop(0, HIDDEN // UNROLL, pass2, jsp0)
            pltpu.async_copy(ob, out_hbm.at[pl.ds(base + g * C, C)], so)
        return carry

    lax.fori_loop(0, NCHUNK // 2, outer, 0)

    for sl in range(2):
        _, _, ob, _, _, so = slots[sl]
        pltpu.make_async_copy(ob, out_hbm.at[pl.ds(base, C)], so).wait()


def kernel(input_ids, token_type_ids, position_ids, word_embeddings,
           position_embeddings, token_type_embeddings, ln_gamma, ln_beta):
    ids = input_ids.reshape(-1).astype(jnp.int32)
    pids = position_ids.reshape(-1).astype(jnp.int32)
    tids = token_type_ids.reshape(-1).astype(jnp.int32)
    cids = pids * TYPE_VOCAB + tids
    ctab = (position_embeddings[:, None, :]
            + token_type_embeddings[None, :, :]).reshape(
                MAX_POS * TYPE_VOCAB, HIDDEN)
    out = _sc_embed(ids, cids, word_embeddings, ctab, ln_gamma, ln_beta)
    return out.reshape(B, S, HIDDEN)


# speedup vs baseline: 7.2428x; 5.9780x over previous
import functools

import jax
import jax.numpy as jnp
from jax import lax
from jax.experimental import pallas as pl
from jax.experimental.pallas import tpu as pltpu
from jax.experimental.pallas import tpu_sc as plsc

VOCAB = 30522
HIDDEN = 768
MAX_POS = 512
TYPE_VOCAB = 2
LN_EPS = 1e-12
B, S = 1024, 512
N = B * S

NC, NS, L = 2, 16, 16
NW = NC * NS
TPW = N // NW
C = 16
NCHUNK = TPW // C
UNROLL = 8

_mesh = plsc.VectorSubcoreMesh(core_axis_name="c", subcore_axis_name="s")


@functools.partial(
    pl.kernel,
    mesh=_mesh,
    out_type=jax.ShapeDtypeStruct((N, HIDDEN), jnp.float32),
    compiler_params=pltpu.CompilerParams(
        use_tc_tiling_on_sc=False, needs_layout_passes=False
    ),
    scratch_types=[
        pltpu.VMEM((TPW,), jnp.int32),
        pltpu.VMEM((TPW,), jnp.int32),
        pltpu.VMEM((HIDDEN,), jnp.float32),
        pltpu.VMEM((HIDDEN,), jnp.float32),
        pltpu.VMEM((C, HIDDEN), jnp.float32),
        pltpu.VMEM((C, HIDDEN), jnp.float32),
        pltpu.VMEM((C, HIDDEN), jnp.float32),
        pltpu.VMEM((C, HIDDEN), jnp.float32),
        pltpu.VMEM((C, HIDDEN), jnp.float32),
        pltpu.VMEM((C, HIDDEN), jnp.float32),
        pltpu.SemaphoreType.DMA,
        pltpu.SemaphoreType.DMA,
        pltpu.SemaphoreType.DMA,
        pltpu.SemaphoreType.DMA,
        pltpu.SemaphoreType.DMA,
        pltpu.SemaphoreType.DMA,
    ],
)
def _sc_embed(ids_hbm, cids_hbm, wtab, ctab, gam, bet,
              out_hbm,
              widx, cidx, gvm, bvm,
              wb0, wb1, cb0, cb1, ob0, ob1,
              sw0, sw1, sc0, sc1, so0, so1):
    wid = lax.axis_index("s") * NC + lax.axis_index("c")
    base = wid * TPW

    pltpu.sync_copy(ids_hbm.at[pl.ds(base, TPW)], widx)
    pltpu.sync_copy(cids_hbm.at[pl.ds(base, TPW)], cidx)
    pltpu.sync_copy(gam, gvm)
    pltpu.sync_copy(bet, bvm)

    lanes = lax.iota(jnp.int32, L)
    slots = ((wb0, cb0, ob0, sw0, sc0, so0),
             (wb1, cb1, ob1, sw1, sc1, so1))

    def issue(g, slot):
        wb, cb, _, sw, sc_, _ = slots[slot]
        c0 = g * C
        wvec = widx[pl.ds(c0, C)]
        cvec = cidx[pl.ds(c0, C)]
        pltpu.async_copy(wtab.at[wvec], wb, sw)
        pltpu.async_copy(ctab.at[cvec], cb, sc_)

    issue(0, 0)

    def outer(g2, carry):
        for sl in range(2):
            wb, cb, ob, sw, sc_, so = slots[sl]
            g = g2 * 2 + sl

            @pl.when(g + 1 < NCHUNK)
            def _():
                issue(g + 1, 1 - sl)

            dummy = widx[pl.ds(0, C)]
            pltpu.make_async_copy(wtab.at[dummy], wb, sw).wait()
            pltpu.make_async_copy(ctab.at[dummy], cb, sc_).wait()

            zero = jnp.zeros((L,), jnp.float32)

            def tok1(c, carry1):
                svec, qvec = carry1

                def blk(jb, acc):
                    s0, q0, s1, q1 = acc
                    for u in range(4):
                        j0 = (jb * 4 + u) * L
                        wv = wb[c, pl.ds(j0, L)]
                        cv = cb[c, pl.ds(j0, L)]
                        x = wv + cv
                        wb[c, pl.ds(j0, L)] = x
                        if u % 2 == 0:
                            s0 = s0 + x
                            q0 = q0 + x * x
                        else:
                            s1 = s1 + x
                            q1 = q1 + x * x
                    return (s0, q0, s1, q1)

                s0, q0, s1, q1 = lax.fori_loop(
                    0, HIDDEN // (4 * L), blk, (zero, zero, zero, zero))
                ssum = jnp.sum(s0 + s1)
                qsum = jnp.sum(q0 + q1)
                svec = jnp.where(lanes == c, ssum, svec)
                qvec = jnp.where(lanes == c, qsum, qvec)
                return (svec, qvec)

            s, q = lax.fori_loop(0, C, tok1, (zero, zero))
            mean = s * (1.0 / HIDDEN)
            var = q * (1.0 / HIDDEN) - mean * mean
            y = var + LN_EPS
            ib = plsc.bitcast(y, jnp.int32)
            ib = 0x5F3759DF - lax.shift_right_logical(ib, 1)
            r = plsc.bitcast(ib, jnp.float32)
            for _ in range(4):
                r = r * (1.5 - 0.5 * y * r * r)
            mr = mean * r

            @pl.when(g >= 2)
            def _():
                pltpu.make_async_copy(
                    ob, out_hbm.at[pl.ds(base, C)], so).wait()

            rb = [jnp.full((L,), r[c], jnp.float32) for c in range(C)]
            mrb = [jnp.full((L,), mr[c], jnp.float32) for c in range(C)]

            def pass2(jb, carry2):
                j0 = jb * L
                gv = gvm[pl.ds(j0, L)]
                bv = bvm[pl.ds(j0, L)]
                for c in range(C):
                    xv = wb[c, pl.ds(j0, L)]
                    yv = (xv * rb[c] - mrb[c]) * gv + bv
                    ob[c, pl.ds(j0, L)] = yv
                return carry2

            lax.fori_loop(0, HIDDEN // L, pass2, 0)
            pltpu.async_copy(ob, out_hbm.at[pl.ds(base + g * C, C)], so)
        return carry

    lax.fori_loop(0, NCHUNK // 2, outer, 0)

    for sl in range(2):
        _, _, ob, _, _, so = slots[sl]
        pltpu.make_async_copy(ob, out_hbm.at[pl.ds(base, C)], so).wait()


def kernel(input_ids, token_type_ids, position_ids, word_embeddings,
           position_embeddings, token_type_embeddings, ln_gamma, ln_beta):
    ids = input_ids.reshape(-1).astype(jnp.int32)
    pids = position_ids.reshape(-1).astype(jnp.int32)
    tids = token_type_ids.reshape(-1).astype(jnp.int32)
    cids = pids * TYPE_VOCAB + tids
    ctab = (position_embeddings[:, None, :]
            + token_type_embeddings[None, :, :]).reshape(
                MAX_POS * TYPE_VOCAB, HIDDEN)
    out = _sc_embed(ids, cids, word_embeddings, ctab, ln_gamma, ln_beta)
    return out.reshape(B, S, HIDDEN)


# speedup vs baseline: 7.4994x; 1.0354x over previous
import functools

import jax
import jax.numpy as jnp
from jax import lax
from jax.experimental import pallas as pl
from jax.experimental.pallas import tpu as pltpu
from jax.experimental.pallas import tpu_sc as plsc

VOCAB = 30522
HIDDEN = 768
MAX_POS = 512
TYPE_VOCAB = 2
LN_EPS = 1e-12
B, S = 1024, 512
N = B * S

NC, NS, L = 2, 16, 16
NW = NC * NS
TPW = N // NW
C = 16
NCHUNK = TPW // C
UNROLL = 8

_mesh = plsc.VectorSubcoreMesh(core_axis_name="c", subcore_axis_name="s")


@functools.partial(
    pl.kernel,
    mesh=_mesh,
    out_type=jax.ShapeDtypeStruct((B, S, HIDDEN), jnp.float32),
    compiler_params=pltpu.CompilerParams(
        use_tc_tiling_on_sc=False, needs_layout_passes=False
    ),
    scratch_types=[
        pltpu.VMEM((TPW,), jnp.int32),
        pltpu.VMEM((TPW,), jnp.int32),
        pltpu.VMEM((HIDDEN,), jnp.float32),
        pltpu.VMEM((HIDDEN,), jnp.float32),
        pltpu.VMEM((C, HIDDEN), jnp.float32),
        pltpu.VMEM((C, HIDDEN), jnp.float32),
        pltpu.VMEM((C, HIDDEN), jnp.float32),
        pltpu.VMEM((C, HIDDEN), jnp.float32),
        pltpu.VMEM((C, HIDDEN), jnp.float32),
        pltpu.VMEM((C, HIDDEN), jnp.float32),
        pltpu.VMEM((C, 17), jnp.float32),
        pltpu.VMEM((C, 17), jnp.float32),
        pltpu.SemaphoreType.DMA,
        pltpu.SemaphoreType.DMA,
        pltpu.SemaphoreType.DMA,
        pltpu.SemaphoreType.DMA,
        pltpu.SemaphoreType.DMA,
        pltpu.SemaphoreType.DMA,
    ],
)
def _sc_embed(ids_hbm, cids_hbm, wtab, ctab, gam, bet,
              out_hbm,
              widx, cidx, gvm, bvm,
              wb0, wb1, cb0, cb1, ob0, ob1, st_s, st_q,
              sw0, sw1, sc0, sc1, so0, so1):
    wid = lax.axis_index("s") * NC + lax.axis_index("c")
    base = wid * TPW

    pltpu.sync_copy(ids_hbm.at[pl.ds(base, TPW)], widx)
    pltpu.sync_copy(cids_hbm.at[pl.ds(base, TPW)], cidx)
    pltpu.sync_copy(gam, gvm)
    pltpu.sync_copy(bet, bvm)

    lanes = lax.iota(jnp.int32, L)
    slots = ((wb0, cb0, ob0, sw0, sc0, so0),
             (wb1, cb1, ob1, sw1, sc1, so1))

    def issue(g, slot):
        wb, cb, _, sw, sc_, _ = slots[slot]
        c0 = g * C
        wvec = widx[pl.ds(c0, C)]
        cvec = cidx[pl.ds(c0, C)]
        pltpu.async_copy(wtab.at[wvec], wb, sw)
        pltpu.async_copy(ctab.at[cvec], cb, sc_)

    issue(0, 0)

    def outer(g2, carry):
        for sl in range(2):
            wb, cb, ob, sw, sc_, so = slots[sl]
            g = g2 * 2 + sl

            @pl.when(g + 1 < NCHUNK)
            def _():
                issue(g + 1, 1 - sl)

            dummy = widx[pl.ds(0, C)]
            pltpu.make_async_copy(wtab.at[dummy], wb, sw).wait()
            pltpu.make_async_copy(ctab.at[dummy], cb, sc_).wait()

            zero = jnp.zeros((L,), jnp.float32)

            def tok1(c, carry1):
                def blk(jb, acc):
                    s0, q0, s1, q1 = acc
                    for u in range(8):
                        j0 = (jb * 8 + u) * L
                        wv = wb[c, pl.ds(j0, L)]
                        cv = cb[c, pl.ds(j0, L)]
                        x = wv + cv
                        wb[c, pl.ds(j0, L)] = x
                        if u % 2 == 0:
                            s0 = s0 + x
                            q0 = q0 + x * x
                        else:
                            s1 = s1 + x
                            q1 = q1 + x * x
                    return (s0, q0, s1, q1)

                s0, q0, s1, q1 = lax.fori_loop(
                    0, HIDDEN // (8 * L), blk, (zero, zero, zero, zero))
                st_s[c, pl.ds(0, L)] = s0 + s1
                st_q[c, pl.ds(0, L)] = q0 + q1
                return carry1

            lax.fori_loop(0, C, tok1, 0)

            s = zero
            q = zero
            for k in range(L):
                ksp = jnp.full((L,), k, jnp.int32)
                s = s + plsc.load_gather(st_s, [lanes, ksp])
                q = q + plsc.load_gather(st_q, [lanes, ksp])
            mean = s * (1.0 / HIDDEN)
            var = q * (1.0 / HIDDEN) - mean * mean
            y = var + LN_EPS
            ib = plsc.bitcast(y, jnp.int32)
            ib = 0x5F3759DF - lax.shift_right_logical(ib, 1)
            r = plsc.bitcast(ib, jnp.float32)
            for _ in range(4):
                r = r * (1.5 - 0.5 * y * r * r)
            mr = mean * r

            @pl.when(g >= 2)
            def _():
                pltpu.make_async_copy(
                    ob, out_hbm.at[0, pl.ds(0, C)], so).wait()

            rb = [jnp.full((L,), r[c], jnp.float32) for c in range(C)]
            mrb = [jnp.full((L,), mr[c], jnp.float32) for c in range(C)]

            def pass2(jb, carry2):
                j0 = jb * L
                gv = gvm[pl.ds(j0, L)]
                bv = bvm[pl.ds(j0, L)]
                for c in range(C):
                    xv = wb[c, pl.ds(j0, L)]
                    yv = (xv * rb[c] - mrb[c]) * gv + bv
                    ob[c, pl.ds(j0, L)] = yv
                return carry2

            lax.fori_loop(0, HIDDEN // L, pass2, 0)
            token0 = base + g * C
            bi = lax.shift_right_logical(token0, 9)
            si = lax.bitwise_and(token0, S - 1)
            pltpu.async_copy(ob, out_hbm.at[bi, pl.ds(si, C)], so)
        return carry

    lax.fori_loop(0, NCHUNK // 2, outer, 0)

    for sl in range(2):
        _, _, ob, _, _, so = slots[sl]
        pltpu.make_async_copy(ob, out_hbm.at[0, pl.ds(0, C)], so).wait()


def kernel(input_ids, token_type_ids, position_ids, word_embeddings,
           position_embeddings, token_type_embeddings, ln_gamma, ln_beta):
    ids = input_ids.reshape(-1).astype(jnp.int32)
    pids = position_ids.reshape(-1).astype(jnp.int32)
    tids = token_type_ids.reshape(-1).astype(jnp.int32)
    cids = pids * TYPE_VOCAB + tids
    ctab = (position_embeddings[:, None, :]
            + token_type_embeddings[None, :, :]).reshape(
                MAX_POS * TYPE_VOCAB, HIDDEN)
    return _sc_embed(ids, cids, word_embeddings, ctab, ln_gamma, ln_beta)


# speedup vs baseline: 8.2373x; 1.0984x over previous
import functools

import jax
import jax.numpy as jnp
from jax import lax
from jax.experimental import pallas as pl
from jax.experimental.pallas import tpu as pltpu
from jax.experimental.pallas import tpu_sc as plsc

VOCAB = 30522
HIDDEN = 768
MAX_POS = 512
TYPE_VOCAB = 2
LN_EPS = 1e-12
B, S = 1024, 512
N = B * S

NC, NS, L = 2, 16, 16
NW = NC * NS
TPW = N // NW
C = 16
NCHUNK = TPW // C
UNROLL = 8

_mesh = plsc.VectorSubcoreMesh(core_axis_name="c", subcore_axis_name="s")


@functools.partial(
    pl.kernel,
    mesh=_mesh,
    out_type=jax.ShapeDtypeStruct((B, S, HIDDEN), jnp.float32),
    compiler_params=pltpu.CompilerParams(needs_layout_passes=False),
    scratch_types=[
        pltpu.VMEM((TPW,), jnp.int32),
        pltpu.VMEM((TPW,), jnp.int32),
        pltpu.VMEM((HIDDEN,), jnp.float32),
        pltpu.VMEM((HIDDEN,), jnp.float32),
        pltpu.VMEM((C, HIDDEN), jnp.float32),
        pltpu.VMEM((C, HIDDEN), jnp.float32),
        pltpu.VMEM((C, HIDDEN), jnp.float32),
        pltpu.VMEM((C, HIDDEN), jnp.float32),
        pltpu.VMEM((C, HIDDEN), jnp.float32),
        pltpu.VMEM((C, HIDDEN), jnp.float32),
        pltpu.VMEM((C, 17), jnp.float32),
        pltpu.VMEM((C, 17), jnp.float32),
        pltpu.SemaphoreType.DMA,
        pltpu.SemaphoreType.DMA,
        pltpu.SemaphoreType.DMA,
        pltpu.SemaphoreType.DMA,
        pltpu.SemaphoreType.DMA,
        pltpu.SemaphoreType.DMA,
    ],
)
def _sc_embed(ids_hbm, cids_hbm, wtab, ctab, gam, bet,
              out_hbm,
              widx, cidx, gvm, bvm,
              wb0, wb1, cb0, cb1, ob0, ob1, st_s, st_q,
              sw0, sw1, sc0, sc1, so0, so1):
    wid = lax.axis_index("s") * NC + lax.axis_index("c")
    base = wid * TPW

    pltpu.sync_copy(ids_hbm.at[pl.ds(base, TPW)], widx)
    pltpu.sync_copy(cids_hbm.at[pl.ds(base, TPW)], cidx)
    pltpu.sync_copy(gam, gvm)
    pltpu.sync_copy(bet, bvm)

    lanes = lax.iota(jnp.int32, L)
    slots = ((wb0, cb0, ob0, sw0, sc0, so0),
             (wb1, cb1, ob1, sw1, sc1, so1))

    def issue(g, slot):
        wb, cb, _, sw, sc_, _ = slots[slot]
        c0 = g * C
        wvec = widx[pl.ds(c0, C)]
        cvec = cidx[pl.ds(c0, C)]
        pltpu.async_copy(wtab.at[wvec], wb, sw)
        pltpu.async_copy(ctab.at[cvec], cb, sc_)

    issue(0, 0)

    def outer(g2, carry):
        for sl in range(2):
            wb, cb, ob, sw, sc_, so = slots[sl]
            g = g2 * 2 + sl

            @pl.when(g + 1 < NCHUNK)
            def _():
                issue(g + 1, 1 - sl)

            dummy = widx[pl.ds(0, C)]
            pltpu.make_async_copy(wtab.at[dummy], wb, sw).wait()
            pltpu.make_async_copy(ctab.at[dummy], cb, sc_).wait()

            zero = jnp.zeros((L,), jnp.float32)

            def tok1(c, carry1):
                def blk(jb, acc):
                    s0, q0, s1, q1 = acc
                    for u in range(8):
                        j0 = (jb * 8 + u) * L
                        wv = wb[c, pl.ds(j0, L)]
                        cv = cb[c, pl.ds(j0, L)]
                        x = wv + cv
                        wb[c, pl.ds(j0, L)] = x
                        if u % 2 == 0:
                            s0 = s0 + x
                            q0 = q0 + x * x
                        else:
                            s1 = s1 + x
                            q1 = q1 + x * x
                    return (s0, q0, s1, q1)

                s0, q0, s1, q1 = lax.fori_loop(
                    0, HIDDEN // (8 * L), blk, (zero, zero, zero, zero))
                st_s[c, pl.ds(0, L)] = s0 + s1
                st_q[c, pl.ds(0, L)] = q0 + q1
                return carry1

            lax.fori_loop(0, C, tok1, 0)

            s = zero
            q = zero
            for k in range(L):
                ksp = jnp.full((L,), k, jnp.int32)
                s = s + plsc.load_gather(st_s, [lanes, ksp])
                q = q + plsc.load_gather(st_q, [lanes, ksp])
            mean = s * (1.0 / HIDDEN)
            var = q * (1.0 / HIDDEN) - mean * mean
            y = var + LN_EPS
            ib = plsc.bitcast(y, jnp.int32)
            ib = 0x5F3759DF - lax.shift_right_logical(ib, 1)
            r = plsc.bitcast(ib, jnp.float32)
            for _ in range(4):
                r = r * (1.5 - 0.5 * y * r * r)
            mr = mean * r

            @pl.when(g >= 2)
            def _():
                pltpu.make_async_copy(
                    ob, out_hbm.at[0, pl.ds(0, C)], so).wait()

            rb = [jnp.full((L,), r[c], jnp.float32) for c in range(C)]
            mrb = [jnp.full((L,), mr[c], jnp.float32) for c in range(C)]

            def pass2(jb, carry2):
                j0 = jb * L
                gv = gvm[pl.ds(j0, L)]
                bv = bvm[pl.ds(j0, L)]
                for c in range(C):
                    xv = wb[c, pl.ds(j0, L)]
                    yv = (xv * rb[c] - mrb[c]) * gv + bv
                    ob[c, pl.ds(j0, L)] = yv
                return carry2

            lax.fori_loop(0, HIDDEN // L, pass2, 0)
            token0 = base + g * C
            bi = lax.shift_right_logical(token0, 9)
            si = pl.multiple_of(lax.bitwise_and(token0, S - 1), C)
            pltpu.async_copy(ob, out_hbm.at[bi, pl.ds(si, C)], so)
        return carry

    lax.fori_loop(0, NCHUNK // 2, outer, 0)

    for sl in range(2):
        _, _, ob, _, _, so = slots[sl]
        pltpu.make_async_copy(ob, out_hbm.at[0, pl.ds(0, C)], so).wait()


def kernel(input_ids, token_type_ids, position_ids, word_embeddings,
           position_embeddings, token_type_embeddings, ln_gamma, ln_beta):
    ids = input_ids.reshape(-1).astype(jnp.int32)
    pids = position_ids.reshape(-1).astype(jnp.int32)
    tids = token_type_ids.reshape(-1).astype(jnp.int32)
    cids = pids * TYPE_VOCAB + tids
    ctab = (position_embeddings[:, None, :]
            + token_type_embeddings[None, :, :]).reshape(
                MAX_POS * TYPE_VOCAB, HIDDEN)
    return _sc_embed(ids, cids, word_embeddings, ctab, ln_gamma, ln_beta)


# speedup vs baseline: 17.5409x; 2.1294x over previous
import functools

import jax
import jax.numpy as jnp
from jax import lax
from jax.experimental import pallas as pl
from jax.experimental.pallas import tpu as pltpu
from jax.experimental.pallas import tpu_sc as plsc

VOCAB = 30522
HIDDEN = 768
MAX_POS = 512
TYPE_VOCAB = 2
LN_EPS = 1e-12
B, S = 1024, 512
N = B * S

NC, NS, L = 2, 16, 16
NW = NC * NS
TPW = N // NW
C = 16
NCHUNK = TPW // C
UNROLL = 8

_mesh = plsc.VectorSubcoreMesh(core_axis_name="c", subcore_axis_name="s")


@functools.partial(
    pl.kernel,
    mesh=_mesh,
    out_type=jax.ShapeDtypeStruct((B, S, HIDDEN), jnp.float32),
    compiler_params=pltpu.CompilerParams(needs_layout_passes=False),
    scratch_types=[
        pltpu.VMEM((TPW,), jnp.int32),
        pltpu.VMEM((TPW,), jnp.int32),
        pltpu.VMEM((HIDDEN,), jnp.float32),
        pltpu.VMEM((HIDDEN,), jnp.float32),
        pltpu.VMEM((C, HIDDEN), jnp.float32),
        pltpu.VMEM((C, HIDDEN), jnp.float32),
        pltpu.VMEM((C, HIDDEN), jnp.float32),
        pltpu.VMEM((C, HIDDEN), jnp.float32),
        pltpu.VMEM((C, HIDDEN), jnp.float32),
        pltpu.VMEM((C, HIDDEN), jnp.float32),
        pltpu.VMEM((C, HIDDEN), jnp.float32),
        pltpu.VMEM((C, 17), jnp.float32),
        pltpu.VMEM((C, 17), jnp.float32),
        pltpu.SemaphoreType.DMA,
        pltpu.SemaphoreType.DMA,
        pltpu.SemaphoreType.DMA,
        pltpu.SemaphoreType.DMA,
        pltpu.SemaphoreType.DMA,
        pltpu.SemaphoreType.DMA,
    ],
)
def _sc_embed(ids_hbm, cids_hbm, wtab, ctab, gam, bet,
              out_hbm,
              widx, cidx, gvm, bvm,
              wb0, wb1, cb0, cb1, ob0, ob1, xb, st_s, st_q,
              sw0, sw1, sc0, sc1, so0, so1):
    wid = lax.axis_index("s") * NC + lax.axis_index("c")
    base = wid * TPW

    pltpu.sync_copy(ids_hbm.at[pl.ds(base, TPW)], widx)
    pltpu.sync_copy(cids_hbm.at[pl.ds(base, TPW)], cidx)
    pltpu.sync_copy(gam, gvm)
    pltpu.sync_copy(bet, bvm)

    lanes = lax.iota(jnp.int32, L)
    slots = ((wb0, cb0, ob0, sw0, sc0, so0),
             (wb1, cb1, ob1, sw1, sc1, so1))

    def issue(g, slot):
        wb, cb, _, sw, sc_, _ = slots[slot]
        c0 = g * C
        wvec = widx[pl.ds(c0, C)]
        cvec = cidx[pl.ds(c0, C)]
        pltpu.async_copy(wtab.at[wvec], wb, sw)
        pltpu.async_copy(ctab.at[cvec], cb, sc_)

    issue(0, 0)

    def outer(g2, carry):
        for sl in range(2):
            wb, cb, ob, sw, sc_, so = slots[sl]
            g = g2 * 2 + sl

            @pl.when(g + 1 < NCHUNK)
            def _():
                issue(g + 1, 1 - sl)

            dummy = widx[pl.ds(0, C)]
            pltpu.make_async_copy(wtab.at[dummy], wb, sw).wait()
            pltpu.make_async_copy(ctab.at[dummy], cb, sc_).wait()

            zero = jnp.zeros((L,), jnp.float32)

            def tok1(c, carry1):
                def blk(jb, acc):
                    s0, q0 = acc
                    offs = [(jb * 8 + u) * L for u in range(8)]
                    ws = [wb[c, pl.ds(o, L)] for o in offs]
                    cs = [cb[c, pl.ds(o, L)] for o in offs]
                    xs = [ws[u] + cs[u] for u in range(8)]
                    for u in range(8):
                        xb[c, pl.ds(offs[u], L)] = xs[u]
                    qs = [x * x for x in xs]
                    s0 = s0 + (((xs[0] + xs[1]) + (xs[2] + xs[3]))
                               + ((xs[4] + xs[5]) + (xs[6] + xs[7])))
                    q0 = q0 + (((qs[0] + qs[1]) + (qs[2] + qs[3]))
                               + ((qs[4] + qs[5]) + (qs[6] + qs[7])))
                    return (s0, q0)

                s0, q0 = lax.fori_loop(
                    0, HIDDEN // (8 * L), blk, (zero, zero))
                st_s[c, pl.ds(0, L)] = s0
                st_q[c, pl.ds(0, L)] = q0
                return carry1

            lax.fori_loop(0, C, tok1, 0)

            s = zero
            q = zero
            for k in range(L):
                ksp = jnp.full((L,), k, jnp.int32)
                s = s + plsc.load_gather(st_s, [lanes, ksp])
                q = q + plsc.load_gather(st_q, [lanes, ksp])
            mean = s * (1.0 / HIDDEN)
            var = q * (1.0 / HIDDEN) - mean * mean
            y = var + LN_EPS
            ib = plsc.bitcast(y, jnp.int32)
            ib = 0x5F3759DF - lax.shift_right_logical(ib, 1)
            r = plsc.bitcast(ib, jnp.float32)
            for _ in range(4):
                r = r * (1.5 - 0.5 * y * r * r)
            mr = mean * r

            @pl.when(g >= 2)
            def _():
                pltpu.make_async_copy(
                    ob, out_hbm.at[0, pl.ds(0, C)], so).wait()

            rb = [jnp.full((L,), r[c], jnp.float32) for c in range(C)]
            mrb = [jnp.full((L,), mr[c], jnp.float32) for c in range(C)]

            def pass2(jb, carry2):
                j0 = jb * L
                gv = gvm[pl.ds(j0, L)]
                bv = bvm[pl.ds(j0, L)]
                xs = [xb[c, pl.ds(j0, L)] for c in range(C)]
                ys = [(xs[c] * rb[c] - mrb[c]) * gv + bv for c in range(C)]
                for c in range(C):
                    ob[c, pl.ds(j0, L)] = ys[c]
                return carry2

            lax.fori_loop(0, HIDDEN // L, pass2, 0)
            token0 = base + g * C
            bi = lax.shift_right_logical(token0, 9)
            si = pl.multiple_of(lax.bitwise_and(token0, S - 1), C)
            pltpu.async_copy(ob, out_hbm.at[bi, pl.ds(si, C)], so)
        return carry

    lax.fori_loop(0, NCHUNK // 2, outer, 0)

    for sl in range(2):
        _, _, ob, _, _, so = slots[sl]
        pltpu.make_async_copy(ob, out_hbm.at[0, pl.ds(0, C)], so).wait()


def kernel(input_ids, token_type_ids, position_ids, word_embeddings,
           position_embeddings, token_type_embeddings, ln_gamma, ln_beta):
    ids = input_ids.reshape(-1).astype(jnp.int32)
    pids = position_ids.reshape(-1).astype(jnp.int32)
    tids = token_type_ids.reshape(-1).astype(jnp.int32)
    cids = pids * TYPE_VOCAB + tids
    ctab = (position_embeddings[:, None, :]
            + token_type_embeddings[None, :, :]).reshape(
                MAX_POS * TYPE_VOCAB, HIDDEN)
    return _sc_embed(ids, cids, word_embeddings, ctab, ln_gamma, ln_beta)
